# Initial kernel scaffold; baseline (speedup 1.0000x reference)
import jax
import jax.numpy as jnp
from jax.experimental import pallas as pl


def kernel(y_pred, y_true, perm_index, lam):
    raise NotImplementedError("write your pallas kernel here")


# speedup vs baseline: 1.1920x; 1.1920x over previous
import jax
import jax.numpy as jnp
from jax.experimental import pallas as pl

_B, _C = 4096, 1000
_BR = 128
_GRID = _B // _BR


def _mixup_ce_body(x_ref, yt_ref, perm_ref, ytfull_ref, lam_ref, out_ref):
    i = pl.program_id(0)
    x = x_ref[:, :]

    m = jnp.max(x, axis=1, keepdims=True)
    s = jnp.sum(jnp.exp(x - m), axis=1, keepdims=True)
    lse = m + jnp.log(s)

    col = jax.lax.broadcasted_iota(jnp.int32, (_BR, _C), 1)
    labels0 = yt_ref[:, :]
    p0 = jnp.sum(jnp.where(col == labels0, x, 0.0), axis=1, keepdims=True)

    permv = perm_ref[:, :]
    rows = jax.lax.broadcasted_iota(jnp.int32, (_BR, _B), 1)
    ytf = ytfull_ref[:, :]
    labels1 = jnp.sum(jnp.where(rows == permv, ytf, 0), axis=1, keepdims=True)
    p1 = jnp.sum(jnp.where(col == labels1, x, 0.0), axis=1, keepdims=True)

    lam = lam_ref[:, :]
    part = (jnp.sum(lse, axis=0, keepdims=True)
            - lam * jnp.sum(p0, axis=0, keepdims=True)
            - (1.0 - lam) * jnp.sum(p1, axis=0, keepdims=True))

    @pl.when(i == 0)
    def _init():
        out_ref[:, :] = jnp.zeros_like(out_ref)

    out_ref[:, :] += part

    @pl.when(i == _GRID - 1)
    def _fin():
        out_ref[:, :] = out_ref[:, :] * (1.0 / _B)


def kernel(y_pred, y_true, perm_index, lam):
    lam_arr = jnp.asarray(lam, jnp.float32).reshape(1, 1)
    yt2 = y_true.reshape(_B, 1)
    perm2 = perm_index.reshape(_B, 1)
    ytfull = y_true.reshape(1, _B)
    out = pl.pallas_call(
        _mixup_ce_body,
        grid=(_GRID,),
        in_specs=[
            pl.BlockSpec((_BR, _C), lambda i: (i, 0)),
            pl.BlockSpec((_BR, 1), lambda i: (i, 0)),
            pl.BlockSpec((_BR, 1), lambda i: (i, 0)),
            pl.BlockSpec((1, _B), lambda i: (0, 0)),
            pl.BlockSpec((1, 1), lambda i: (0, 0)),
        ],
        out_specs=pl.BlockSpec((1, 1), lambda i: (0, 0)),
        out_shape=jax.ShapeDtypeStruct((1, 1), jnp.float32),
    )(y_pred, yt2, perm2, ytfull, lam_arr)
    return out.reshape(())
